# baseline (speedup 1.0000x reference)
import jax
import jax.numpy as jnp
from jax import lax
from jax.experimental import pallas as pl
from jax.experimental.pallas import tpu as pltpu
from jax.experimental.pallas import tpu_sc as plsc

VOCAB = 21128
HIDDEN = 768
MAX_POS = 512
B = 128
S = 512
LN_EPS = 1e-12

NW = 32
SCHUNK = 8
NSC = S // SCHUNK
NBUF = 2
HALF_B = B // 2


def _make_sc_gather(nseq):
    seq_per_w = nseq // NW
    tpw = seq_per_w * S

    def body(ids_hbm, word_hbm, out_hbm, *scratch):
        rows = scratch[:seq_per_w]
        ids_v, gsem, osem = scratch[seq_per_w:]
        cid = lax.axis_index("c")
        sid = lax.axis_index("s")
        wid = sid * 2 + cid
        tok0 = wid * tpw

        pltpu.sync_copy(ids_hbm.at[pl.ds(tok0, tpw)], ids_v)

        def issue_gathers(c, buf):
            for b in range(seq_per_w):
                idx = ids_v.at[pl.ds(b * S + c * SCHUNK, SCHUNK)]
                pltpu.async_copy(word_hbm.at[idx], rows[b].at[buf],
                                 gsem.at[buf])

        def drain_gathers(buf):
            dummy = word_hbm.at[pl.ds(0, SCHUNK)]
            for b in range(seq_per_w):
                pltpu.make_async_copy(dummy, rows[b].at[buf],
                                      gsem.at[buf]).wait()

        def issue_outs(c, buf):
            for b in range(seq_per_w):
                dst = out_hbm.at[pl.ds(tok0 + b * S + c * SCHUNK, SCHUNK)]
                pltpu.async_copy(rows[b].at[buf], dst, osem.at[buf])

        def drain_outs(buf):
            dummy = word_hbm.at[pl.ds(0, SCHUNK)]
            for b in range(seq_per_w):
                pltpu.make_async_copy(dummy, rows[b].at[buf],
                                      osem.at[buf]).wait()

        issue_gathers(0, 0)

        def chunk_body(c, _):
            buf = lax.rem(c, NBUF)
            obuf = 1 - buf
            drain_gathers(buf)

            @pl.when(c >= 1)
            def _():
                drain_outs(obuf)

            @pl.when(c + 1 < NSC)
            def _():
                issue_gathers(c + 1, obuf)

            issue_outs(c, buf)
            return 0

        lax.fori_loop(0, NSC, chunk_body, 0)
        drain_outs((NSC - 1) % NBUF)

    return pl.kernel(
        body,
        out_type=jax.ShapeDtypeStruct((nseq * S, HIDDEN), jnp.float32),
        mesh=plsc.VectorSubcoreMesh(core_axis_name="c",
                                    subcore_axis_name="s"),
        compiler_params=pltpu.CompilerParams(needs_layout_passes=False),
        scratch_types=(
            [pltpu.VMEM((NBUF, SCHUNK, HIDDEN), jnp.float32)] * seq_per_w
            + [
                pltpu.VMEM((tpw,), jnp.int32),
                pltpu.SemaphoreType.DMA((NBUF,)),
                pltpu.SemaphoreType.DMA((NBUF,)),
            ]
        ),
    )


_sc_gather_full = _make_sc_gather(B)


def _tc_body(emb_ref, ttf_ref, pos_ref, type_ref, gam_ref, bet_ref, o_ref):
    x = emb_ref[0]
    ttf = ttf_ref[0, 0, :]
    d01 = (type_ref[1, :] - type_ref[0, :])[None, :]
    x = x + pos_ref[...] + type_ref[0, :][None, :] + ttf[:, None] * d01
    mean = jnp.mean(x, axis=1, keepdims=True)
    xc = x - mean
    var = jnp.mean(xc * xc, axis=1, keepdims=True)
    y = xc * jax.lax.rsqrt(var + LN_EPS)
    o_ref[0] = y * gam_ref[...] + bet_ref[...]


_tc_ln_lo = pl.pallas_call(
    _tc_body,
    grid=(HALF_B,),
    in_specs=[
        pl.BlockSpec((1, S, HIDDEN), lambda i: (i, 0, 0)),
        pl.BlockSpec((1, 1, S), lambda i: (i, 0, 0)),
        pl.BlockSpec((S, HIDDEN), lambda i: (0, 0)),
        pl.BlockSpec((2, HIDDEN), lambda i: (0, 0)),
        pl.BlockSpec((1, HIDDEN), lambda i: (0, 0)),
        pl.BlockSpec((1, HIDDEN), lambda i: (0, 0)),
    ],
    out_specs=pl.BlockSpec((1, S, HIDDEN), lambda i: (i, 0, 0)),
    out_shape=jax.ShapeDtypeStruct((B, S, HIDDEN), jnp.float32),
)


def _tc_body_hi(emb_ref, ttf_ref, pos_ref, type_ref, gam_ref, bet_ref,
                acc_ref, o_ref):
    del acc_ref
    _tc_body(emb_ref, ttf_ref, pos_ref, type_ref, gam_ref, bet_ref, o_ref)


_tc_ln_hi = pl.pallas_call(
    _tc_body_hi,
    grid=(HALF_B,),
    in_specs=[
        pl.BlockSpec((1, S, HIDDEN), lambda i: (i, 0, 0)),
        pl.BlockSpec((1, 1, S), lambda i: (i, 0, 0)),
        pl.BlockSpec((S, HIDDEN), lambda i: (0, 0)),
        pl.BlockSpec((2, HIDDEN), lambda i: (0, 0)),
        pl.BlockSpec((1, HIDDEN), lambda i: (0, 0)),
        pl.BlockSpec((1, HIDDEN), lambda i: (0, 0)),
        pl.BlockSpec(memory_space=pl.ANY),
    ],
    out_specs=pl.BlockSpec((1, S, HIDDEN), lambda i: (i + HALF_B, 0, 0)),
    out_shape=jax.ShapeDtypeStruct((B, S, HIDDEN), jnp.float32),
    input_output_aliases={6: 0},
)


@jax.jit
def kernel(input_ids, token_type_ids, word_embeddings, position_embeddings,
           token_type_embeddings, ln_gamma, ln_beta):
    ids = input_ids.reshape(-1).astype(jnp.int32)
    emb = _sc_gather_full(ids, word_embeddings).reshape(B, S, HIDDEN)
    ttf = token_type_ids.astype(jnp.float32).reshape(B, 1, S)
    gam = ln_gamma.reshape(1, HIDDEN)
    bet = ln_beta.reshape(1, HIDDEN)
    out0 = _tc_ln_lo(emb[:HALF_B], ttf[:HALF_B],
                     position_embeddings, token_type_embeddings, gam, bet)
    out = _tc_ln_hi(emb[HALF_B:], ttf[HALF_B:],
                    position_embeddings, token_type_embeddings, gam, bet,
                    out0)
    return out


# speedup vs baseline: 1.3559x; 1.3559x over previous
import jax
import jax.numpy as jnp
from jax import lax
from jax.experimental import pallas as pl
from jax.experimental.pallas import tpu as pltpu
from jax.experimental.pallas import tpu_sc as plsc

VOCAB = 21128
HIDDEN = 768
MAX_POS = 512
B = 128
S = 512
LN_EPS = 1e-12

NW = 32
SCHUNK = 8
NSC = S // SCHUNK
NBUF = 2
HALF_B = B // 2


def _make_sc_gather():
    seq_per_w = B // NW
    tpw = seq_per_w * S
    half_tok = HALF_B * S

    def body(ids_hbm, word_hbm, out0_hbm, out1_hbm, *scratch):
        rows = scratch[:seq_per_w]
        ids_v, gsem, osem = scratch[seq_per_w:]
        cid = lax.axis_index("c")
        sid = lax.axis_index("s")
        wid = sid * 2 + cid
        tok0 = wid * tpw
        in_hi = tok0 >= half_tok
        loc0 = tok0 - lax.select(in_hi, half_tok, 0)

        pltpu.sync_copy(ids_hbm.at[pl.ds(tok0, tpw)], ids_v)

        def issue_gathers(c, buf):
            for b in range(seq_per_w):
                idx = ids_v.at[pl.ds(b * S + c * SCHUNK, SCHUNK)]
                pltpu.async_copy(word_hbm.at[idx], rows[b].at[buf],
                                 gsem.at[buf])

        def drain_gathers(buf):
            dummy = word_hbm.at[pl.ds(0, SCHUNK)]
            for b in range(seq_per_w):
                pltpu.make_async_copy(dummy, rows[b].at[buf],
                                      gsem.at[buf]).wait()

        def issue_outs(c, buf):
            @pl.when(jnp.logical_not(in_hi))
            def _():
                for b in range(seq_per_w):
                    dst = out0_hbm.at[
                        pl.ds(loc0 + b * S + c * SCHUNK, SCHUNK)]
                    pltpu.async_copy(rows[b].at[buf], dst, osem.at[buf])

            @pl.when(in_hi)
            def _():
                for b in range(seq_per_w):
                    dst = out1_hbm.at[
                        pl.ds(loc0 + b * S + c * SCHUNK, SCHUNK)]
                    pltpu.async_copy(rows[b].at[buf], dst, osem.at[buf])

        def drain_outs(buf):
            dummy = word_hbm.at[pl.ds(0, SCHUNK)]
            for b in range(seq_per_w):
                pltpu.make_async_copy(dummy, rows[b].at[buf],
                                      osem.at[buf]).wait()

        issue_gathers(0, 0)

        def chunk_body(c, _):
            buf = lax.rem(c, NBUF)
            obuf = 1 - buf
            drain_gathers(buf)

            @pl.when(c >= 1)
            def _():
                drain_outs(obuf)

            @pl.when(c + 1 < NSC)
            def _():
                issue_gathers(c + 1, obuf)

            issue_outs(c, buf)
            return 0

        lax.fori_loop(0, NSC, chunk_body, 0)
        drain_outs((NSC - 1) % NBUF)

    return pl.kernel(
        body,
        out_type=(
            jax.ShapeDtypeStruct((HALF_B * S, HIDDEN), jnp.float32),
            jax.ShapeDtypeStruct((HALF_B * S, HIDDEN), jnp.float32),
        ),
        mesh=plsc.VectorSubcoreMesh(core_axis_name="c",
                                    subcore_axis_name="s"),
        compiler_params=pltpu.CompilerParams(needs_layout_passes=False),
        scratch_types=(
            [pltpu.VMEM((NBUF, SCHUNK, HIDDEN), jnp.float32)] * seq_per_w
            + [
                pltpu.VMEM((tpw,), jnp.int32),
                pltpu.SemaphoreType.DMA((NBUF,)),
                pltpu.SemaphoreType.DMA((NBUF,)),
            ]
        ),
    )


_sc_gather_split = _make_sc_gather()


def _tc_body(emb_ref, ttf_ref, pos_ref, type_ref, gam_ref, bet_ref, o_ref):
    x = emb_ref[0]
    ttf = ttf_ref[0, 0, :]
    d01 = (type_ref[1, :] - type_ref[0, :])[None, :]
    x = x + pos_ref[...] + type_ref[0, :][None, :] + ttf[:, None] * d01
    mean = jnp.mean(x, axis=1, keepdims=True)
    xc = x - mean
    var = jnp.mean(xc * xc, axis=1, keepdims=True)
    y = xc * jax.lax.rsqrt(var + LN_EPS)
    o_ref[0] = y * gam_ref[...] + bet_ref[...]


_tc_ln_lo = pl.pallas_call(
    _tc_body,
    grid=(HALF_B,),
    in_specs=[
        pl.BlockSpec((1, S, HIDDEN), lambda i: (i, 0, 0)),
        pl.BlockSpec((1, 1, S), lambda i: (i, 0, 0)),
        pl.BlockSpec((S, HIDDEN), lambda i: (0, 0)),
        pl.BlockSpec((2, HIDDEN), lambda i: (0, 0)),
        pl.BlockSpec((1, HIDDEN), lambda i: (0, 0)),
        pl.BlockSpec((1, HIDDEN), lambda i: (0, 0)),
    ],
    out_specs=pl.BlockSpec((1, S, HIDDEN), lambda i: (i, 0, 0)),
    out_shape=jax.ShapeDtypeStruct((B, S, HIDDEN), jnp.float32),
)


def _tc_body_hi(emb_ref, ttf_ref, pos_ref, type_ref, gam_ref, bet_ref,
                acc_ref, o_ref):
    del acc_ref
    _tc_body(emb_ref, ttf_ref, pos_ref, type_ref, gam_ref, bet_ref, o_ref)


_tc_ln_hi = pl.pallas_call(
    _tc_body_hi,
    grid=(HALF_B,),
    in_specs=[
        pl.BlockSpec((1, S, HIDDEN), lambda i: (i, 0, 0)),
        pl.BlockSpec((1, 1, S), lambda i: (i, 0, 0)),
        pl.BlockSpec((S, HIDDEN), lambda i: (0, 0)),
        pl.BlockSpec((2, HIDDEN), lambda i: (0, 0)),
        pl.BlockSpec((1, HIDDEN), lambda i: (0, 0)),
        pl.BlockSpec((1, HIDDEN), lambda i: (0, 0)),
        pl.BlockSpec(memory_space=pl.ANY),
    ],
    out_specs=pl.BlockSpec((1, S, HIDDEN), lambda i: (i + HALF_B, 0, 0)),
    out_shape=jax.ShapeDtypeStruct((B, S, HIDDEN), jnp.float32),
    input_output_aliases={6: 0},
)


@jax.jit
def kernel(input_ids, token_type_ids, word_embeddings, position_embeddings,
           token_type_embeddings, ln_gamma, ln_beta):
    ids = input_ids.reshape(-1).astype(jnp.int32)
    emb0, emb1 = _sc_gather_split(ids, word_embeddings)
    ttf = token_type_ids.astype(jnp.float32).reshape(B, 1, S)
    gam = ln_gamma.reshape(1, HIDDEN)
    bet = ln_beta.reshape(1, HIDDEN)
    out0 = _tc_ln_lo(emb0.reshape(HALF_B, S, HIDDEN), ttf[:HALF_B],
                     position_embeddings, token_type_embeddings, gam, bet)
    out = _tc_ln_hi(emb1.reshape(HALF_B, S, HIDDEN), ttf[HALF_B:],
                    position_embeddings, token_type_embeddings, gam, bet,
                    out0)
    return out


# speedup vs baseline: 1.3670x; 1.0081x over previous
import jax
import jax.numpy as jnp
from jax import lax
from jax.experimental import pallas as pl
from jax.experimental.pallas import tpu as pltpu
from jax.experimental.pallas import tpu_sc as plsc

VOCAB = 21128
HIDDEN = 768
MAX_POS = 512
B = 128
S = 512
LN_EPS = 1e-12

NW = 32
SEQ_PER_W = B // NW
TPW = SEQ_PER_W * S
SCHUNK = 8
NSC = S // SCHUNK
NBUF = 2


def _sc_body(ids_hbm, word_hbm, out_hbm, rows0, rows1, rows2, rows3,
             ids_v, gsem, osem):
    rows = (rows0, rows1, rows2, rows3)
    cid = lax.axis_index("c")
    sid = lax.axis_index("s")
    wid = sid * 2 + cid
    tok0 = wid * TPW

    pltpu.sync_copy(ids_hbm.at[pl.ds(tok0, TPW)], ids_v)

    def issue_gathers(c, buf):
        for b in range(SEQ_PER_W):
            idx = ids_v.at[pl.ds(b * S + c * SCHUNK, SCHUNK)]
            pltpu.async_copy(word_hbm.at[idx], rows[b].at[buf],
                             gsem.at[buf])

    def drain_gathers(buf):
        dummy = word_hbm.at[pl.ds(0, SCHUNK)]
        for b in range(SEQ_PER_W):
            pltpu.make_async_copy(dummy, rows[b].at[buf],
                                  gsem.at[buf]).wait()

    def issue_outs(c, buf):
        for b in range(SEQ_PER_W):
            dst = out_hbm.at[pl.ds(tok0 + b * S + c * SCHUNK, SCHUNK)]
            pltpu.async_copy(rows[b].at[buf], dst, osem.at[buf])

    def drain_outs(buf):
        dummy = word_hbm.at[pl.ds(0, SCHUNK)]
        for b in range(SEQ_PER_W):
            pltpu.make_async_copy(dummy, rows[b].at[buf],
                                  osem.at[buf]).wait()

    issue_gathers(0, 0)

    def chunk_body(c, _):
        buf = lax.rem(c, NBUF)
        obuf = 1 - buf
        drain_gathers(buf)

        @pl.when(c >= 1)
        def _():
            drain_outs(obuf)

        @pl.when(c + 1 < NSC)
        def _():
            issue_gathers(c + 1, obuf)

        issue_outs(c, buf)
        return 0

    lax.fori_loop(0, NSC, chunk_body, 0)
    drain_outs((NSC - 1) % NBUF)


_sc_gather = pl.kernel(
    _sc_body,
    out_type=jax.ShapeDtypeStruct((B * S, HIDDEN), jnp.float32),
    mesh=plsc.VectorSubcoreMesh(core_axis_name="c", subcore_axis_name="s"),
    compiler_params=pltpu.CompilerParams(needs_layout_passes=False),
    scratch_types=[
        pltpu.VMEM((NBUF, SCHUNK, HIDDEN), jnp.float32),
        pltpu.VMEM((NBUF, SCHUNK, HIDDEN), jnp.float32),
        pltpu.VMEM((NBUF, SCHUNK, HIDDEN), jnp.float32),
        pltpu.VMEM((NBUF, SCHUNK, HIDDEN), jnp.float32),
        pltpu.VMEM((TPW,), jnp.int32),
        pltpu.SemaphoreType.DMA((NBUF,)),
        pltpu.SemaphoreType.DMA((NBUF,)),
    ],
)


def _tc_body(emb_ref, ttf_ref, pos_ref, type_ref, gam_ref, bet_ref, o_ref):
    x = emb_ref[0]
    ttf = ttf_ref[0, 0, :]
    d01 = (type_ref[1, :] - type_ref[0, :])[None, :]
    x = x + pos_ref[...] + type_ref[0, :][None, :] + ttf[:, None] * d01
    mean = jnp.mean(x, axis=1, keepdims=True)
    xc = x - mean
    var = jnp.mean(xc * xc, axis=1, keepdims=True)
    y = xc * jax.lax.rsqrt(var + LN_EPS)
    o_ref[0] = y * gam_ref[...] + bet_ref[...]


_tc_ln = pl.pallas_call(
    _tc_body,
    grid=(B,),
    in_specs=[
        pl.BlockSpec((1, S, HIDDEN), lambda i: (i, 0, 0)),
        pl.BlockSpec((1, 1, S), lambda i: (i, 0, 0)),
        pl.BlockSpec((S, HIDDEN), lambda i: (0, 0)),
        pl.BlockSpec((2, HIDDEN), lambda i: (0, 0)),
        pl.BlockSpec((1, HIDDEN), lambda i: (0, 0)),
        pl.BlockSpec((1, HIDDEN), lambda i: (0, 0)),
    ],
    out_specs=pl.BlockSpec((1, S, HIDDEN), lambda i: (i, 0, 0)),
    out_shape=jax.ShapeDtypeStruct((B, S, HIDDEN), jnp.float32),
)


@jax.jit
def kernel(input_ids, token_type_ids, word_embeddings, position_embeddings,
           token_type_embeddings, ln_gamma, ln_beta):
    ids = input_ids.reshape(-1).astype(jnp.int32)
    emb = _sc_gather(ids, word_embeddings)
    ttf = token_type_ids.astype(jnp.float32).reshape(B, 1, S)
    out = _tc_ln(emb.reshape(B, S, HIDDEN), ttf, position_embeddings,
                 token_type_embeddings, ln_gamma.reshape(1, HIDDEN),
                 ln_beta.reshape(1, HIDDEN))
    return out
